# baseline (speedup 1.0000x reference)
import functools

import jax
import jax.numpy as jnp
import numpy as np
from jax import lax
from jax.experimental import pallas as pl
from jax.experimental.pallas import tpu as pltpu
from jax.experimental.pallas import tpu_sc as plsc

_ANCHORS = np.array(
    [[1.3221, 1.73145], [3.19275, 4.00944], [5.05587, 8.09892],
     [9.47112, 4.84053], [11.2364, 10.0071]], dtype=np.float32)
_THRESH = 0.6
_GH, _GW = 16, 32
_NA, _NCLS = 5, 7
_B, _T = 64, 50
_CELLS = _NA * _GH * _GW
_HW = _GH * _GW
_SC_CORES, _SC_SUBCORES = 2, 16
_NW = _SC_CORES * _SC_SUBCORES
_TPAD = 256


def _sc_body(tgt_hbm, mask_o, conf_o, tx_o, ty_o, rw_o, rh_o, lab_o,
             tgt_v0, tgt_v1,
             m0_t, c0_t, x0_t, y0_t, w0_t, h0_t, l0_t,
             m1_t, c1_t, x1_t, y1_t, w1_t, h1_t, l1_t, sem):
  wid = lax.axis_index("s") * _SC_CORES + lax.axis_index("c")
  b0 = wid
  b1 = wid + _NW
  lanes = lax.iota(jnp.int32, 16)
  in5 = lanes < _NA
  cl5 = jnp.where(in5, lanes, 0)
  zf = jnp.zeros((16,), jnp.float32)
  zi = jnp.zeros((16,), jnp.int32)

  def _lanes_const(vals):
    v = zf + 1.0
    for i, c in enumerate(vals):
      v = jnp.where(lanes == i, float(c), v)
    return v

  aw = _lanes_const(_ANCHORS[:, 0])
  ah = _lanes_const(_ANCHORS[:, 1])
  awah = aw * ah

  set0 = (m0_t, c0_t, x0_t, y0_t, w0_t, h0_t, l0_t)
  set1 = (m1_t, c1_t, x1_t, y1_t, w1_t, h1_t, l1_t)

  pend = [pltpu.async_copy(tgt_hbm.at[b0], tgt_v0, sem),
          pltpu.async_copy(tgt_hbm.at[b1], tgt_v1, sem)]

  def init_body(i, c):
    sl = pl.ds(i * 16, 16)
    for (mt, ct, _, _, _, _, lt) in (set0, set1):
      mt[sl] = zf
      ct[sl] = zf + 1.0
      lt[sl] = zf + 127.0
    return c
  lax.fori_loop(0, _CELLS // 16, init_body, 0)
  for h in pend:
    h.wait()

  def ifloor(v):
    r = v.astype(jnp.int32)
    return r - (r.astype(jnp.float32) > v).astype(jnp.int32)

  def process(t, tv, mt, ct, xt, yt, wt, ht, lt):
    idx = t * 5 + cl5
    row = plsc.load_gather(tv, [idx], mask=in5)
    row = jnp.where(in5, row, 0.0)

    def bc(k):
      return row.at[zi + k].get(mode="promise_in_bounds")

    labv = bc(0)
    xv = bc(1)
    yv = bc(2)
    wv = bc(3)
    hv = bc(4)
    valid_v = (labv + xv + yv + wv + hv) != 0.0
    gx = xv * float(_GW)
    gy = yv * float(_GH)
    gwv = wv * float(_GW)
    ghv = hv * float(_GH)
    giv = ifloor(gx)
    gjv = ifloor(gy)
    labq = ifloor(labv).astype(jnp.float32)
    inter = jnp.minimum(gwv, aw) * jnp.minimum(ghv, ah)
    union = gwv * ghv + awah - inter + 1e-16
    iou = jnp.where(in5, inter / union, -1.0)
    cmx = plsc.cummax(iou)
    mxv = cmx.at[zi + 15].get(mode="promise_in_bounds")
    eq = iou == mxv
    first = jnp.cumsum(eq.astype(jnp.int32)) == 1
    is_best = eq & first
    high = iou > _THRESH
    cellv = gjv * _GW + giv
    idxv = cl5 * _HW + cellv

    plsc.store_scatter(ct, [idxv], jnp.where(is_best, 1.0, 0.0),
                       mask=valid_v & (high | is_best))
    wm = is_best & valid_v
    plsc.store_scatter(mt, [idxv], zf + 1.0, mask=wm)
    plsc.store_scatter(xt, [idxv], gx - giv.astype(jnp.float32), mask=wm)
    plsc.store_scatter(yt, [idxv], gy - gjv.astype(jnp.float32), mask=wm)
    plsc.store_scatter(wt, [idxv], gwv / aw, mask=wm)
    plsc.store_scatter(ht, [idxv], ghv / ah, mask=wm)
    old = plsc.load_gather(lt, [idxv], mask=wm)
    plsc.store_scatter(lt, [idxv], jnp.minimum(old, labq), mask=wm)

  def t_body(t, c):
    process(t, tgt_v0, *set0)
    process(t, tgt_v1, *set1)
    return c
  lax.fori_loop(0, _T, t_body, 0)

  outs = (mask_o, conf_o, tx_o, ty_o, rw_o, rh_o, lab_o)
  pend = []
  for b, tabs in ((b0, set0), (b1, set1)):
    for o, tab in zip(outs, tabs):
      pend.append(pltpu.async_copy(tab, o.at[b], sem))
  for h in pend:
    h.wait()


@jax.jit
def _sc_build(tgt_pad):
  tab = jax.ShapeDtypeStruct((_B, _CELLS), jnp.float32)
  f = pl.kernel(
      _sc_body,
      out_type=(tab,) * 7,
      mesh=plsc.VectorSubcoreMesh(core_axis_name="c", subcore_axis_name="s",
                                  num_cores=_SC_CORES,
                                  num_subcores=_SC_SUBCORES),
      scratch_types=[pltpu.VMEM((_TPAD,), jnp.float32)] * 2 +
                    [pltpu.VMEM((_CELLS,), jnp.float32)] * 14 +
                    [pltpu.SemaphoreType.DMA],
      compiler_params=pltpu.CompilerParams(needs_layout_passes=False),
  )
  return f(tgt_pad)


def _sigmoid(z):
  return 1.0 / (1.0 + jnp.exp(-z))


_BCHUNK = 16


def _tc_body(x_ref, mask_ref, conf_ref, tx_ref, ty_ref, rw_ref, rh_ref,
             lab_ref, out_ref, acc_ref):
  b = pl.program_id(0)

  @pl.when(b == 0)
  def _init():
    for i in range(6):
      acc_ref[i] = 0.0

  obj_s = 0.0
  bce_noobj = 0.0
  n_obj = 0.0
  n_noobj = 0.0
  def xr(ch):
    return x_ref[:, ch, :].astype(jnp.float32)

  for a in range(_NA):
    sl = pl.ds(a * _HW, _HW)
    mask_v = mask_ref[:, sl]
    conf_v = conf_ref[:, sl]
    obj = mask_v > 0.5
    noobj_v = jnp.where(conf_v != mask_v, 1.0, 0.0)

    px = _sigmoid(xr(a * 14 + 0))
    py = _sigmoid(xr(a * 14 + 1))
    pw = xr(a * 14 + 2)
    ph = xr(a * 14 + 3)
    zconf = xr(a * 14 + 6)

    twv = jnp.log(rw_ref[:, sl] + 1e-16)
    thv = jnp.log(rh_ref[:, sl] + 1e-16)
    d = ((px - tx_ref[:, sl]) ** 2 + (py - ty_ref[:, sl]) ** 2 +
         (pw - twv) ** 2 + (ph - thv) ** 2)

    sp = jnp.maximum(zconf, 0.0) + jnp.log(1.0 + jnp.exp(-jnp.abs(zconf)))
    bce = sp - mask_v * zconf
    bce_noobj = bce_noobj + noobj_v * bce

    s = [_sigmoid(xr(a * 14 + 7 + cc)) for cc in range(_NCLS)]
    sumexp = jnp.exp(s[0])
    for cc in range(1, _NCLS):
      sumexp += jnp.exp(s[cc])
    lse = jnp.log(sumexp)
    lab_v = lab_ref[:, sl]
    picked = -lse
    for cc in range(_NCLS):
      picked += jnp.where(lab_v == float(cc), s[cc], 0.0)

    obj_s = obj_s + jnp.where(obj, d + bce - (1.0 / float(_B)) * picked, 0.0)
    n_obj = n_obj + mask_v
    n_noobj = n_noobj + noobj_v

  acc_ref[0] += jnp.sum(obj_s)
  acc_ref[2] += jnp.sum(bce_noobj)
  acc_ref[4] += jnp.sum(n_obj)
  acc_ref[5] += jnp.sum(n_noobj)

  @pl.when(b == pl.num_programs(0) - 1)
  def _fin():
    no = acc_ref[4]
    nn = acc_ref[5]
    out_ref[0, 0] = acc_ref[0] / no + acc_ref[2] / nn


@jax.jit
def _tc_loss(x3, mask, conf, tx, ty, rw, rh, lab):
  tab_spec = pl.BlockSpec((_BCHUNK, _CELLS), lambda b: (b, 0))
  return pl.pallas_call(
      _tc_body,
      grid=(_B // _BCHUNK,),
      in_specs=[pl.BlockSpec((_BCHUNK, 14 * _NA, _HW), lambda b: (b, 0, 0))] +
               [tab_spec] * 7,
      out_specs=pl.BlockSpec(memory_space=pltpu.SMEM),
      out_shape=jax.ShapeDtypeStruct((1, 1), jnp.float32),
      scratch_shapes=[pltpu.SMEM((6,), jnp.float32)],
  )(x3, mask, conf, tx, ty, rw, rh, lab)


@jax.jit
def kernel(x, targets):
  tgt_pad = jnp.pad(jnp.reshape(targets, (_B, _T * 5)),
                    ((0, 0), (0, _TPAD - _T * 5)))
  tabs = _sc_build(tgt_pad)
  x3 = jnp.reshape(x, (_B, 14 * _NA, _HW)).astype(jnp.bfloat16)
  out = _tc_loss(x3, *tabs)
  return jnp.reshape(out, ())


# speedup vs baseline: 1.5192x; 1.5192x over previous
import functools

import jax
import jax.numpy as jnp
import numpy as np
from jax import lax
from jax.experimental import pallas as pl
from jax.experimental.pallas import tpu as pltpu
from jax.experimental.pallas import tpu_sc as plsc

_ANCHORS = np.array(
    [[1.3221, 1.73145], [3.19275, 4.00944], [5.05587, 8.09892],
     [9.47112, 4.84053], [11.2364, 10.0071]], dtype=np.float32)
_THRESH = 0.6
_GH, _GW = 16, 32
_NA, _NCLS = 5, 7
_B, _T = 64, 50
_CELLS = _NA * _GH * _GW
_HW = _GH * _GW
_SC_CORES, _SC_SUBCORES = 2, 16
_NW = _SC_CORES * _SC_SUBCORES
_TPAD = 256


def _sc_body(tgt_hbm, mask_o, conf_o, tx_o, ty_o, rw_o, rh_o, lab_o,
             tgt_v0, tgt_v1,
             m0_t, c0_t, x0_t, y0_t, w0_t, h0_t, l0_t,
             m1_t, c1_t, x1_t, y1_t, w1_t, h1_t, l1_t, sem):
  wid = lax.axis_index("s") * _SC_CORES + lax.axis_index("c")
  b0 = wid
  b1 = wid + _NW
  lanes = lax.iota(jnp.int32, 16)
  in5 = lanes < _NA
  cl5 = jnp.where(in5, lanes, 0)
  zf = jnp.zeros((16,), jnp.float32)
  zi = jnp.zeros((16,), jnp.int32)

  def _lanes_const(vals):
    v = zf + 1.0
    for i, c in enumerate(vals):
      v = jnp.where(lanes == i, float(c), v)
    return v

  aw = _lanes_const(_ANCHORS[:, 0])
  ah = _lanes_const(_ANCHORS[:, 1])
  awah = aw * ah

  set0 = (m0_t, c0_t, x0_t, y0_t, w0_t, h0_t, l0_t)
  set1 = (m1_t, c1_t, x1_t, y1_t, w1_t, h1_t, l1_t)

  pend = [pltpu.async_copy(tgt_hbm.at[b0], tgt_v0, sem),
          pltpu.async_copy(tgt_hbm.at[b1], tgt_v1, sem)]

  def init_body(i, c):
    sl = pl.ds(i * 16, 16)
    for (mt, ct, _, _, _, _, lt) in (set0, set1):
      mt[sl] = zf
      ct[sl] = zf + 1.0
      lt[sl] = zf + 127.0
    return c
  lax.fori_loop(0, _CELLS // 16, init_body, 0)
  for h in pend:
    h.wait()

  def ifloor(v):
    r = v.astype(jnp.int32)
    return r - (r.astype(jnp.float32) > v).astype(jnp.int32)

  def process(t, tv, mt, ct, xt, yt, wt, ht, lt):
    idx = t * 5 + cl5
    row = plsc.load_gather(tv, [idx], mask=in5)
    row = jnp.where(in5, row, 0.0)

    def bc(k):
      return row.at[zi + k].get(mode="promise_in_bounds")

    labv = bc(0)
    xv = bc(1)
    yv = bc(2)
    wv = bc(3)
    hv = bc(4)
    valid_v = (labv + xv + yv + wv + hv) != 0.0
    gx = xv * float(_GW)
    gy = yv * float(_GH)
    gwv = wv * float(_GW)
    ghv = hv * float(_GH)
    giv = ifloor(gx)
    gjv = ifloor(gy)
    labq = ifloor(labv).astype(jnp.float32)
    inter = jnp.minimum(gwv, aw) * jnp.minimum(ghv, ah)
    union = gwv * ghv + awah - inter + 1e-16
    iou = jnp.where(in5, inter / union, -1.0)
    cmx = plsc.cummax(iou)
    mxv = cmx.at[zi + 15].get(mode="promise_in_bounds")
    eq = iou == mxv
    first = jnp.cumsum(eq.astype(jnp.int32)) == 1
    is_best = eq & first
    high = iou > _THRESH
    cellv = gjv * _GW + giv
    idxv = cl5 * _HW + cellv

    plsc.store_scatter(ct, [idxv], jnp.where(is_best, 1.0, 0.0),
                       mask=valid_v & (high | is_best))
    wm = is_best & valid_v
    plsc.store_scatter(mt, [idxv], zf + 1.0, mask=wm)
    plsc.store_scatter(xt, [idxv], gx - giv.astype(jnp.float32), mask=wm)
    plsc.store_scatter(yt, [idxv], gy - gjv.astype(jnp.float32), mask=wm)
    plsc.store_scatter(wt, [idxv], gwv / aw, mask=wm)
    plsc.store_scatter(ht, [idxv], ghv / ah, mask=wm)
    old = plsc.load_gather(lt, [idxv], mask=wm)
    plsc.store_scatter(lt, [idxv], jnp.minimum(old, labq), mask=wm)

  def t_body(t, c):
    process(t, tgt_v0, *set0)
    process(t, tgt_v1, *set1)
    return c
  lax.fori_loop(0, _T, t_body, 0)

  outs = (mask_o, conf_o, tx_o, ty_o, rw_o, rh_o, lab_o)
  pend = []
  for b, tabs in ((b0, set0), (b1, set1)):
    for o, tab in zip(outs, tabs):
      pend.append(pltpu.async_copy(tab, o.at[b], sem))
  for h in pend:
    h.wait()


@jax.jit
def _sc_build(tgt_pad):
  tab = jax.ShapeDtypeStruct((_B, _CELLS), jnp.float32)
  f = pl.kernel(
      _sc_body,
      out_type=(tab,) * 7,
      mesh=plsc.VectorSubcoreMesh(core_axis_name="c", subcore_axis_name="s",
                                  num_cores=_SC_CORES,
                                  num_subcores=_SC_SUBCORES),
      scratch_types=[pltpu.VMEM((_TPAD,), jnp.float32)] * 2 +
                    [pltpu.VMEM((_CELLS,), jnp.float32)] * 14 +
                    [pltpu.SemaphoreType.DMA],
      compiler_params=pltpu.CompilerParams(needs_layout_passes=False),
  )
  return f(tgt_pad)


def _sigmoid(z):
  return 1.0 / (1.0 + jnp.exp(-z))


_BCHUNK = 16


def _tc_body(x_ref, mask_ref, conf_ref, tx_ref, ty_ref, rw_ref, rh_ref,
             lab_ref, out_ref, acc_ref):
  b = pl.program_id(0)

  @pl.when(b == 0)
  def _init():
    for i in range(6):
      acc_ref[i] = 0.0

  obj_s = 0.0
  bce_noobj = 0.0
  n_obj = 0.0
  n_noobj = 0.0
  def xr(ch):
    return x_ref[:, ch, :].astype(jnp.float32)

  for a in range(_NA):
    sl = pl.ds(a * _HW, _HW)
    mask_v = mask_ref[:, sl]
    conf_v = conf_ref[:, sl]
    obj = mask_v > 0.5
    noobj_v = jnp.where(conf_v != mask_v, 1.0, 0.0)

    px = _sigmoid(xr(a * 14 + 0))
    py = _sigmoid(xr(a * 14 + 1))
    pw = xr(a * 14 + 2)
    ph = xr(a * 14 + 3)
    zconf = xr(a * 14 + 6)

    twv = jnp.log(rw_ref[:, sl] + 1e-16)
    thv = jnp.log(rh_ref[:, sl] + 1e-16)
    d = ((px - tx_ref[:, sl]) ** 2 + (py - ty_ref[:, sl]) ** 2 +
         (pw - twv) ** 2 + (ph - thv) ** 2)

    sp = jnp.maximum(zconf, 0.0) + jnp.log(1.0 + jnp.exp(-jnp.abs(zconf)))
    bce = sp - mask_v * zconf
    bce_noobj = bce_noobj + noobj_v * bce

    s = [_sigmoid(xr(a * 14 + 7 + cc)) for cc in range(_NCLS)]
    sumexp = jnp.exp(s[0])
    for cc in range(1, _NCLS):
      sumexp += jnp.exp(s[cc])
    lse = jnp.log(sumexp)
    lab_v = lab_ref[:, sl]
    picked = -lse
    for cc in range(_NCLS):
      picked += jnp.where(lab_v == float(cc), s[cc], 0.0)

    obj_s = obj_s + jnp.where(obj, d + bce - (1.0 / float(_B)) * picked, 0.0)
    n_obj = n_obj + mask_v
    n_noobj = n_noobj + noobj_v

  acc_ref[0] += jnp.sum(obj_s)
  acc_ref[2] += jnp.sum(bce_noobj)
  acc_ref[4] += jnp.sum(n_obj)
  acc_ref[5] += jnp.sum(n_noobj)

  @pl.when(b == pl.num_programs(0) - 1)
  def _fin():
    no = acc_ref[4]
    nn = acc_ref[5]
    out_ref[0, 0] = acc_ref[0] / no + acc_ref[2] / nn


@jax.jit
def _tc_loss(x3, mask, conf, tx, ty, rw, rh, lab):
  tab_spec = pl.BlockSpec((_BCHUNK, _CELLS), lambda b: (b, 0))
  return pl.pallas_call(
      _tc_body,
      grid=(_B // _BCHUNK,),
      in_specs=[pl.BlockSpec((_BCHUNK, 14 * _NA, _HW), lambda b: (b, 0, 0))] +
               [tab_spec] * 7,
      out_specs=pl.BlockSpec(memory_space=pltpu.SMEM),
      out_shape=jax.ShapeDtypeStruct((1, 1), jnp.float32),
      scratch_shapes=[pltpu.SMEM((6,), jnp.float32)],
  )(x3, mask, conf, tx, ty, rw, rh, lab)


@jax.jit
def kernel(x, targets):
  tgt_pad = jnp.pad(jnp.reshape(targets, (_B, _T * 5)),
                    ((0, 0), (0, _TPAD - _T * 5)))
  tabs = _sc_build(tgt_pad)
  x3 = jnp.reshape(x, (_B, 14 * _NA, _HW))
  out = _tc_loss(x3, *tabs)
  return jnp.reshape(out, ())
